# baseline (speedup 1.0000x reference)
import functools

import jax
import jax.numpy as jnp
from jax import lax
from jax.experimental import pallas as pl
from jax.experimental.pallas import tpu as pltpu
from jax.experimental.pallas import tpu_sc as plsc

N_NODES = 10000
N_EDGES = 320000
NFEAT = 128
NCLASS = 64
BN_EPS = 1e-5

_NC = 2
_NS = 16
_EDGES_PER_TILE = N_EDGES // (_NC * _NS)
_T = 125
_NT = _EDGES_PER_TILE // _T
_HALFT = _NT // 2
_TPB = 4
_NBODYH = (_HALFT - _TPB) // _TPB
_NBUF = 2
_ROWS_MAIN = 624
_ROWS_LAST = N_NODES - 15 * _ROWS_MAIN


def _sc_segment_sum(x, e4):
    mesh = plsc.VectorSubcoreMesh(core_axis_name="c", subcore_axis_name="s")

    @functools.partial(
        pl.kernel,
        mesh=mesh,
        out_type=jax.ShapeDtypeStruct((_NC * N_NODES, NFEAT), jnp.float32),
        scratch_types=[
            pltpu.VMEM((_HALFT, _T), jnp.int32),
            pltpu.VMEM((_HALFT, _T), jnp.int32),
            pltpu.VMEM((_NBUF, _T, NFEAT), jnp.float32),
            pltpu.VMEM_SHARED((N_NODES, NFEAT), jnp.float32),
            pltpu.SemaphoreType.DMA,
            pltpu.SemaphoreType.DMA,
            pltpu.SemaphoreType.DMA,
            pltpu.SemaphoreType.DMA,
        ],
    )
    def k(x_hbm, e_hbm, out_hbm, sidx, didx, rows, agg_sh,
          gsem_0, gsem_1, ssem_0, ssem_1):
        c = lax.axis_index("c")
        s = lax.axis_index("s")
        wid = c * _NS + s
        row0 = s * _ROWS_MAIN
        gsem = (gsem_0, gsem_1)
        ssem = (ssem_0, ssem_1)

        def issue_gather(k, b):
            pltpu.async_copy(x_hbm.at[sidx.at[k]], rows.at[b], gsem[b])

        def wait_gather(b):
            pltpu.make_async_copy(x_hbm.at[sidx.at[0]], rows.at[b],
                                  gsem[b]).wait()

        def issue_scatter(k, b):
            pltpu.async_copy(rows.at[b], agg_sh.at[didx.at[k]], ssem[b],
                             add=True)

        def wait_scatter(b):
            pltpu.make_async_copy(rows.at[b], agg_sh.at[didx.at[0]],
                                  ssem[b]).wait()

        def body(j, carry):
            k0 = j * _TPB
            for t in range(_TPB):
                k = k0 + t
                b = t % _NBUF
                wait_gather(b)
                issue_scatter(k, b)
                wait_scatter(b)
                issue_gather(k + _NBUF, b)
            return carry

        def do_half():
            issue_gather(0, 0)
            issue_gather(1, 1)
            lax.fori_loop(0, _NBODYH, body, 0)
            for kk in range(_NBODYH * _TPB, _HALFT):
                b = kk % _NBUF
                wait_gather(b)
                issue_scatter(kk, b)
                if kk + _NBUF < _HALFT:
                    wait_scatter(b)
                    issue_gather(kk + _NBUF, b)
            for kk in range(_HALFT - _NBUF, _HALFT):
                wait_scatter(kk % _NBUF)

        ih_s = pltpu.async_copy(e_hbm.at[0, wid, 0], sidx, gsem_0)
        ih_d = pltpu.async_copy(e_hbm.at[1, wid, 0], didx, gsem_1)

        @pl.when(s < _NS - 1)
        def _():
            pltpu.sync_copy(x_hbm.at[pl.ds(row0, _ROWS_MAIN)],
                            agg_sh.at[pl.ds(row0, _ROWS_MAIN)])

        @pl.when(s == _NS - 1)
        def _():
            pltpu.sync_copy(x_hbm.at[pl.ds(row0, _ROWS_LAST)],
                            agg_sh.at[pl.ds(row0, _ROWS_LAST)])

        ih_s.wait()
        ih_d.wait()
        plsc.subcore_barrier()
        do_half()
        pltpu.sync_copy(e_hbm.at[0, wid, 1], sidx)
        pltpu.sync_copy(e_hbm.at[1, wid, 1], didx)
        do_half()
        plsc.subcore_barrier()

        @pl.when(s < _NS - 1)
        def _():
            pltpu.sync_copy(agg_sh.at[pl.ds(row0, _ROWS_MAIN)],
                            out_hbm.at[pl.ds(c * N_NODES + row0, _ROWS_MAIN)])

        @pl.when(s == _NS - 1)
        def _():
            pltpu.sync_copy(agg_sh.at[pl.ds(row0, _ROWS_LAST)],
                            out_hbm.at[pl.ds(c * N_NODES + row0, _ROWS_LAST)])

    return k(x, e4)


_MB = 1000
_NB = N_NODES // _MB


def _tc_mlp(p, x, W1, b1, gamma, beta, W2, b2, Wfc, bfc):
    def body(p_ref, x_ref, w1_ref, b1_ref, g_ref, be_ref, w2_ref, b2_ref,
             wfc_ref, bfc_ref, o_ref, h1_ref, st_ref):
        i = pl.program_id(0)

        @pl.when(i < _NB)
        def _():
            h = p_ref[0] + p_ref[1] - x_ref[...]
            h = jnp.dot(h, w1_ref[...], preferred_element_type=jnp.float32)
            h = jnp.maximum(h + b1_ref[...], 0.0)
            h1_ref[pl.ds(i * _MB, _MB), :] = h
            ssum = jnp.sum(h, axis=0, keepdims=True)
            ssq = jnp.sum(h * h, axis=0, keepdims=True)

            @pl.when(i == 0)
            def _():
                st_ref[0:1, :] = ssum
                st_ref[1:2, :] = ssq

            @pl.when(i > 0)
            def _():
                st_ref[0:1, :] += ssum
                st_ref[1:2, :] += ssq

        @pl.when(i >= _NB)
        def _():
            j = i - _NB
            mean = st_ref[0:1, :] * (1.0 / N_NODES)
            var = st_ref[1:2, :] * (1.0 / N_NODES) - mean * mean
            scale = g_ref[...] * jax.lax.rsqrt(var + BN_EPS)
            shift = be_ref[...] - mean * scale
            h = h1_ref[pl.ds(j * _MB, _MB), :] * scale + shift
            h = jnp.dot(h, w2_ref[...], preferred_element_type=jnp.float32) + b2_ref[...]
            o_ref[...] = (jnp.dot(h, wfc_ref[...],
                                  preferred_element_type=jnp.float32)
                          + bfc_ref[...])

    p3 = p.reshape(2, N_NODES, NFEAT)
    blk = lambda i: jnp.minimum(i, _NB - 1)
    return pl.pallas_call(
        body,
        grid=(2 * _NB,),
        in_specs=[
            pl.BlockSpec((2, _MB, NFEAT), lambda i: (0, blk(i), 0)),
            pl.BlockSpec((_MB, NFEAT), lambda i: (blk(i), 0)),
            pl.BlockSpec((NFEAT, NFEAT), lambda i: (0, 0)),
            pl.BlockSpec((1, NFEAT), lambda i: (0, 0)),
            pl.BlockSpec((1, NFEAT), lambda i: (0, 0)),
            pl.BlockSpec((1, NFEAT), lambda i: (0, 0)),
            pl.BlockSpec((NFEAT, NFEAT), lambda i: (0, 0)),
            pl.BlockSpec((1, NFEAT), lambda i: (0, 0)),
            pl.BlockSpec((NFEAT, NCLASS), lambda i: (0, 0)),
            pl.BlockSpec((1, NCLASS), lambda i: (0, 0)),
        ],
        out_specs=pl.BlockSpec((_MB, NCLASS),
                               lambda i: (jnp.maximum(i - _NB, 0), 0)),
        out_shape=jax.ShapeDtypeStruct((N_NODES, NCLASS), jnp.float32),
        scratch_shapes=[
            pltpu.VMEM((N_NODES, NFEAT), jnp.float32),
            pltpu.VMEM((2, NFEAT), jnp.float32),
        ],
    )(p3, x, W1, b1.reshape(1, -1), gamma.reshape(1, -1), beta.reshape(1, -1),
      W2, b2.reshape(1, -1), Wfc, bfc.reshape(1, -1))


def kernel(x, edge_index, W1, b1, gamma, beta, W2, b2, Wfc, bfc):
    e4 = edge_index.reshape(2, _NC * _NS, 2, _HALFT, _T)
    p = _sc_segment_sum(x, e4)
    return _tc_mlp(p, x, W1, b1, gamma, beta, W2, b2, Wfc, bfc)


# speedup vs baseline: 1.0267x; 1.0267x over previous
import functools

import jax
import jax.numpy as jnp
from jax import lax
from jax.experimental import pallas as pl
from jax.experimental.pallas import tpu as pltpu
from jax.experimental.pallas import tpu_sc as plsc

N_NODES = 10000
N_EDGES = 320000
NFEAT = 128
NCLASS = 64
BN_EPS = 1e-5

_NC = 2
_NS = 16
_EDGES_PER_TILE = N_EDGES // (_NC * _NS)
_T = 80
_NT = _EDGES_PER_TILE // _T
_SLAB = _EDGES_PER_TILE + 112
_TPB = 4
_NBODY = 30
_NBUF = 2
_ROWS_MAIN = 624
_ROWS_LAST = N_NODES - 15 * _ROWS_MAIN


def _sc_segment_sum(x, e4):
    mesh = plsc.VectorSubcoreMesh(core_axis_name="c", subcore_axis_name="s")

    @functools.partial(
        pl.kernel,
        mesh=mesh,
        out_type=jax.ShapeDtypeStruct((_NC * N_NODES, NFEAT), jnp.float32),
        scratch_types=[
            pltpu.VMEM((_SLAB,), jnp.int32),
            pltpu.VMEM((_SLAB,), jnp.int32),
            pltpu.VMEM((_NBUF, _T, NFEAT), jnp.float32),
            pltpu.VMEM_SHARED((N_NODES, NFEAT), jnp.float32),
            pltpu.SemaphoreType.DMA,
            pltpu.SemaphoreType.DMA,
            pltpu.SemaphoreType.DMA,
            pltpu.SemaphoreType.DMA,
        ],
    )
    def k(x_hbm, e_hbm, out_hbm, sidx, didx, rows, agg_sh,
          gsem_0, gsem_1, ssem_0, ssem_1):
        c = lax.axis_index("c")
        s = lax.axis_index("s")
        wid = c * _NS + s
        row0 = s * _ROWS_MAIN
        gsem = (gsem_0, gsem_1)
        ssem = (ssem_0, ssem_1)
        m = lax.rem(16 * wid, 128)
        abase = pl.multiple_of(_EDGES_PER_TILE * wid - m, 128)

        def idx_at(ref, k):
            return ref.at[pl.ds(pl.multiple_of(m + k * _T, 16), _T)]

        def issue_gather(k, b):
            pltpu.async_copy(x_hbm.at[idx_at(sidx, k)], rows.at[b], gsem[b])

        def wait_gather(b):
            pltpu.make_async_copy(x_hbm.at[idx_at(sidx, 0)], rows.at[b],
                                  gsem[b]).wait()

        def issue_scatter(k, b):
            pltpu.async_copy(rows.at[b], agg_sh.at[idx_at(didx, k)], ssem[b],
                             add=True)

        def wait_scatter(b):
            pltpu.make_async_copy(rows.at[b], agg_sh.at[idx_at(didx, 0)],
                                  ssem[b]).wait()

        def body(j, carry):
            k0 = j * _TPB
            for t in range(_TPB):
                k = k0 + t
                b = t % _NBUF
                wait_gather(b)
                issue_scatter(k, b)
                wait_scatter(b)
                issue_gather(k + _NBUF, b)
            return carry

        ih_s = pltpu.async_copy(e_hbm.at[0, pl.ds(abase, _SLAB)], sidx,
                                gsem_0)
        ih_d = pltpu.async_copy(e_hbm.at[1, pl.ds(abase, _SLAB)], didx,
                                gsem_1)

        @pl.when(s < _NS - 1)
        def _():
            pltpu.sync_copy(x_hbm.at[pl.ds(row0, _ROWS_MAIN)],
                            agg_sh.at[pl.ds(row0, _ROWS_MAIN)])

        @pl.when(s == _NS - 1)
        def _():
            pltpu.sync_copy(x_hbm.at[pl.ds(row0, _ROWS_LAST)],
                            agg_sh.at[pl.ds(row0, _ROWS_LAST)])

        ih_s.wait()
        ih_d.wait()
        plsc.subcore_barrier()
        issue_gather(0, 0)
        issue_gather(1, 1)
        lax.fori_loop(0, _NBODY, body, 0)
        for kk in range(_NBODY * _TPB, _NT):
            b = kk % _NBUF
            wait_gather(b)
            issue_scatter(kk, b)
            if kk + _NBUF < _NT:
                wait_scatter(b)
                issue_gather(kk + _NBUF, b)
        for kk in range(_NT - _NBUF, _NT):
            wait_scatter(kk % _NBUF)
        plsc.subcore_barrier()

        @pl.when(s < _NS - 1)
        def _():
            pltpu.sync_copy(agg_sh.at[pl.ds(row0, _ROWS_MAIN)],
                            out_hbm.at[pl.ds(c * N_NODES + row0, _ROWS_MAIN)])

        @pl.when(s == _NS - 1)
        def _():
            pltpu.sync_copy(agg_sh.at[pl.ds(row0, _ROWS_LAST)],
                            out_hbm.at[pl.ds(c * N_NODES + row0, _ROWS_LAST)])

    return k(x, e4)


def _tc_mlp(p, x, W1, b1, gamma, beta, W2, b2, Wfc, bfc):
    def body(p_ref, x_ref, w1_ref, b1_ref, g_ref, be_ref, w2_ref, b2_ref,
             wfc_ref, bfc_ref, o_ref):
        h = p_ref[0:N_NODES, :] + p_ref[N_NODES:2 * N_NODES, :] - x_ref[...]
        h = jnp.dot(h, w1_ref[...], preferred_element_type=jnp.float32) + b1_ref[...]
        h = jnp.maximum(h, 0.0)
        mean = jnp.mean(h, axis=0, keepdims=True)
        d = h - mean
        var = jnp.mean(d * d, axis=0, keepdims=True)
        h = d * (g_ref[...] * jax.lax.rsqrt(var + BN_EPS)) + be_ref[...]
        h = jnp.dot(h, w2_ref[...], preferred_element_type=jnp.float32) + b2_ref[...]
        o_ref[...] = (jnp.dot(h, wfc_ref[...], preferred_element_type=jnp.float32)
                      + bfc_ref[...])

    return pl.pallas_call(
        body,
        out_shape=jax.ShapeDtypeStruct((N_NODES, NCLASS), jnp.float32),
    )(p, x, W1, b1.reshape(1, -1), gamma.reshape(1, -1), beta.reshape(1, -1),
      W2, b2.reshape(1, -1), Wfc, bfc.reshape(1, -1))


def kernel(x, edge_index, W1, b1, gamma, beta, W2, b2, Wfc, bfc):
    p = _sc_segment_sum(x, edge_index)
    return _tc_mlp(p, x, W1, b1, gamma, beta, W2, b2, Wfc, bfc)


# speedup vs baseline: 1.0976x; 1.0691x over previous
import functools

import jax
import jax.numpy as jnp
from jax import lax
from jax.experimental import pallas as pl
from jax.experimental.pallas import tpu as pltpu
from jax.experimental.pallas import tpu_sc as plsc

N_NODES = 10000
N_EDGES = 320000
NFEAT = 128
NCLASS = 64
BN_EPS = 1e-5

_NC = 2
_NS = 16
_EDGES_PER_TILE = N_EDGES // (_NC * _NS)
_T = 112
_NTF = 89
_TAIL = _EDGES_PER_TILE - _NTF * _T
_SLAB = _EDGES_PER_TILE + 112
_TPB = 4
_NBODY = 21
_NBUF = 2
_ROWS_MAIN = 624
_ROWS_LAST = N_NODES - 15 * _ROWS_MAIN


def _sc_segment_sum(x, e4):
    mesh = plsc.VectorSubcoreMesh(core_axis_name="c", subcore_axis_name="s")

    @functools.partial(
        pl.kernel,
        mesh=mesh,
        out_type=jax.ShapeDtypeStruct((_NC * N_NODES, NFEAT), jnp.float32),
        scratch_types=[
            pltpu.VMEM((_SLAB,), jnp.int32),
            pltpu.VMEM((_SLAB,), jnp.int32),
            pltpu.VMEM((_NBUF, _T, NFEAT), jnp.float32),
            pltpu.VMEM_SHARED((N_NODES, NFEAT), jnp.float32),
            pltpu.SemaphoreType.DMA,
            pltpu.SemaphoreType.DMA,
            pltpu.SemaphoreType.DMA,
            pltpu.SemaphoreType.DMA,
        ],
    )
    def k(x_hbm, e_hbm, out_hbm, sidx, didx, rows, agg_sh,
          gsem_0, gsem_1, ssem_0, ssem_1):
        c = lax.axis_index("c")
        s = lax.axis_index("s")
        wid = c * _NS + s
        row0 = s * _ROWS_MAIN
        gsem = (gsem_0, gsem_1)
        ssem = (ssem_0, ssem_1)
        m = lax.rem(16 * wid, 128)
        abase = pl.multiple_of(_EDGES_PER_TILE * wid - m, 128)

        def idx_at(ref, k):
            return ref.at[pl.ds(pl.multiple_of(m + k * _T, 16), _T)]

        def issue_gather(k, b):
            pltpu.async_copy(x_hbm.at[idx_at(sidx, k)], rows.at[b], gsem[b])

        def wait_gather(b):
            pltpu.make_async_copy(x_hbm.at[idx_at(sidx, 0)], rows.at[b],
                                  gsem[b]).wait()

        def issue_scatter(k, b):
            pltpu.async_copy(rows.at[b], agg_sh.at[idx_at(didx, k)], ssem[b],
                             add=True)

        def wait_scatter(b):
            pltpu.make_async_copy(rows.at[b], agg_sh.at[idx_at(didx, 0)],
                                  ssem[b]).wait()

        def body(j, carry):
            k0 = j * _TPB
            for t in range(_TPB):
                k = k0 + t
                b = t % _NBUF
                wait_gather(b)
                issue_scatter(k, b)
                wait_scatter(b)
                issue_gather(k + _NBUF, b)
            return carry

        ih_s = pltpu.async_copy(e_hbm.at[0, pl.ds(abase, _SLAB)], sidx,
                                gsem_0)
        ih_d = pltpu.async_copy(e_hbm.at[1, pl.ds(abase, _SLAB)], didx,
                                gsem_1)

        @pl.when(s < _NS - 1)
        def _():
            pltpu.sync_copy(x_hbm.at[pl.ds(row0, _ROWS_MAIN)],
                            agg_sh.at[pl.ds(row0, _ROWS_MAIN)])

        @pl.when(s == _NS - 1)
        def _():
            pltpu.sync_copy(x_hbm.at[pl.ds(row0, _ROWS_LAST)],
                            agg_sh.at[pl.ds(row0, _ROWS_LAST)])

        tail_off = pl.multiple_of(m + _NTF * _T, 16)

        def tail_gather_desc():
            return pltpu.make_async_copy(
                x_hbm.at[sidx.at[pl.ds(tail_off, _TAIL)]],
                rows.at[_NTF % _NBUF, pl.ds(0, _TAIL)], gsem[_NTF % _NBUF])

        def tail_scatter_desc():
            return pltpu.make_async_copy(
                rows.at[_NTF % _NBUF, pl.ds(0, _TAIL)],
                agg_sh.at[didx.at[pl.ds(tail_off, _TAIL)]],
                ssem[_NTF % _NBUF])

        ih_s.wait()
        ih_d.wait()
        plsc.subcore_barrier()
        issue_gather(0, 0)
        issue_gather(1, 1)
        lax.fori_loop(0, _NBODY, body, 0)
        for kk in range(_NBODY * _TPB, _NTF):
            b = kk % _NBUF
            wait_gather(b)
            issue_scatter(kk, b)
            if kk + _NBUF < _NTF:
                wait_scatter(b)
                issue_gather(kk + _NBUF, b)
            elif kk + _NBUF == _NTF:
                wait_scatter(b)
                pltpu.async_copy(
                    x_hbm.at[sidx.at[pl.ds(tail_off, _TAIL)]],
                    rows.at[_NTF % _NBUF, pl.ds(0, _TAIL)],
                    gsem[_NTF % _NBUF])
        tail_gather_desc().wait()
        pltpu.async_copy(rows.at[_NTF % _NBUF, pl.ds(0, _TAIL)],
                         agg_sh.at[didx.at[pl.ds(tail_off, _TAIL)]],
                         ssem[_NTF % _NBUF], add=True)
        wait_scatter((_NTF - 1) % _NBUF)
        tail_scatter_desc().wait()
        plsc.subcore_barrier()

        @pl.when(s < _NS - 1)
        def _():
            pltpu.sync_copy(agg_sh.at[pl.ds(row0, _ROWS_MAIN)],
                            out_hbm.at[pl.ds(c * N_NODES + row0, _ROWS_MAIN)])

        @pl.when(s == _NS - 1)
        def _():
            pltpu.sync_copy(agg_sh.at[pl.ds(row0, _ROWS_LAST)],
                            out_hbm.at[pl.ds(c * N_NODES + row0, _ROWS_LAST)])

    return k(x, e4)


def _tc_mlp(p, x, W1, b1, gamma, beta, W2, b2, Wfc, bfc):
    def body(p_ref, x_ref, w1_ref, b1_ref, g_ref, be_ref, w2_ref, b2_ref,
             wfc_ref, bfc_ref, o_ref):
        h = p_ref[0:N_NODES, :] + p_ref[N_NODES:2 * N_NODES, :] - x_ref[...]
        h = jnp.dot(h, w1_ref[...], preferred_element_type=jnp.float32) + b1_ref[...]
        h = jnp.maximum(h, 0.0)
        mean = jnp.mean(h, axis=0, keepdims=True)
        d = h - mean
        var = jnp.mean(d * d, axis=0, keepdims=True)
        h = d * (g_ref[...] * jax.lax.rsqrt(var + BN_EPS)) + be_ref[...]
        h = jnp.dot(h, w2_ref[...], preferred_element_type=jnp.float32) + b2_ref[...]
        o_ref[...] = (jnp.dot(h, wfc_ref[...], preferred_element_type=jnp.float32)
                      + bfc_ref[...])

    return pl.pallas_call(
        body,
        out_shape=jax.ShapeDtypeStruct((N_NODES, NCLASS), jnp.float32),
    )(p, x, W1, b1.reshape(1, -1), gamma.reshape(1, -1), beta.reshape(1, -1),
      W2, b2.reshape(1, -1), Wfc, bfc.reshape(1, -1))


def kernel(x, edge_index, W1, b1, gamma, beta, W2, b2, Wfc, bfc):
    p = _sc_segment_sum(x, edge_index)
    return _tc_mlp(p, x, W1, b1, gamma, beta, W2, b2, Wfc, bfc)
